# baseline (speedup 1.0000x reference)
import jax
import jax.numpy as jnp
from jax import lax
from jax.experimental import pallas as pl
from jax.experimental.pallas import tpu as pltpu
from jax.experimental.pallas import tpu_sc as plsc

B = 16384
F = 100
FP = 128
V = 1000000
NC = 2
NS = 16
NW = NC * NS
ROWS_PER_W = B // NW
RBLK = 128
NBLK = ROWS_PER_W // RBLK
GW = 104


def _sc_body(idx_hbm, val_hbm, w_hbm, bias_hbm, out_hbm,
             idx_v, val_v, gat_v, out_v, bias_v, drain_v, sem):
    wid = lax.axis_index("s") * NC + lax.axis_index("c")
    pltpu.sync_copy(bias_hbm, bias_v)
    lane = jax.lax.iota(jnp.int32, 16)
    tail_mask = lane >= 12

    def block(blk, carry):
        row0 = wid * ROWS_PER_W + blk * RBLK
        pltpu.sync_copy(idx_hbm.at[pl.ds(row0, RBLK)], idx_v)
        pltpu.sync_copy(val_hbm.at[pl.ds(row0, RBLK)], val_v)

        def issue(r, c):
            pltpu.async_copy(w_hbm.at[idx_v.at[r, pl.ds(0, F)]],
                             gat_v.at[r, pl.ds(0, F)], sem)
            return c

        lax.fori_loop(0, RBLK, issue, 0)
        pltpu.make_async_copy(w_hbm.at[pl.ds(0, RBLK * F)],
                              drain_v, sem).wait()

        bvec = bias_v[...]

        def sixteen_rows(rg, c):
            r0 = rg * 16
            outv = jnp.zeros((16,), jnp.float32)
            for k in range(16):
                r = r0 + k
                acc = (gat_v[r, pl.ds(0, 16)] * val_v[r, pl.ds(0, 16)]
                       + gat_v[r, pl.ds(16, 16)] * val_v[r, pl.ds(16, 16)])
                acc2 = (gat_v[r, pl.ds(32, 16)] * val_v[r, pl.ds(32, 16)]
                        + gat_v[r, pl.ds(48, 16)] * val_v[r, pl.ds(48, 16)])
                acc3 = (gat_v[r, pl.ds(64, 16)] * val_v[r, pl.ds(64, 16)]
                        + gat_v[r, pl.ds(80, 16)] * val_v[r, pl.ds(80, 16)])
                tail = jnp.where(tail_mask,
                                 gat_v[r, pl.ds(84, 16)]
                                 * val_v[r, pl.ds(84, 16)], 0.0)
                s = jnp.sum(acc + acc2 + acc3 + tail)
                outv = jnp.where(lane == k, s, outv)
            out_v[pl.ds(r0, 16)] = outv + bvec
            return c

        lax.fori_loop(0, RBLK // 16, sixteen_rows, 0)
        pltpu.sync_copy(out_v, out_hbm.at[pl.ds(row0, RBLK)])
        return carry

    lax.fori_loop(0, NBLK, block, 0)


@jax.jit
def _sc_call(idx_p, val_p, w0, bias16):
    mesh = plsc.VectorSubcoreMesh(core_axis_name="c", subcore_axis_name="s")
    f = pl.kernel(
        _sc_body,
        mesh=mesh,
        out_type=jax.ShapeDtypeStruct((B,), jnp.float32),
        scratch_types=[
            pltpu.VMEM((RBLK, FP), jnp.int32),
            pltpu.VMEM((RBLK, FP), jnp.float32),
            pltpu.VMEM((RBLK, GW), jnp.float32),
            pltpu.VMEM((RBLK,), jnp.float32),
            pltpu.VMEM((16,), jnp.float32),
            pltpu.VMEM((RBLK * F,), jnp.float32),
            pltpu.SemaphoreType.DMA,
        ],
        compiler_params=pltpu.CompilerParams(needs_layout_passes=False),
    )
    return f(idx_p, val_p, w0, bias16)


def kernel(index_list, value_list, W, bias):
    idx_p = jax.lax.dynamic_update_slice(
        jnp.zeros((B, FP), index_list.dtype), index_list, (0, 0))
    val_p = jax.lax.dynamic_update_slice(
        jnp.zeros((B, FP), value_list.dtype), value_list, (0, 0))
    w0 = W[0]
    bias16 = jnp.broadcast_to(bias, (16,))
    res = _sc_call(idx_p, val_p, w0, bias16)
    return res.reshape(B, 1)


# speedup vs baseline: 1.2137x; 1.2137x over previous
import jax
import jax.numpy as jnp
from jax import lax
from jax.experimental import pallas as pl
from jax.experimental.pallas import tpu as pltpu
from jax.experimental.pallas import tpu_sc as plsc

B = 16384
F = 100
FP = 128
V = 1000000
NC = 2
NS = 16
NW = NC * NS
ROWS_PER_W = B // NW
RBLK = 128
NBLK = ROWS_PER_W // RBLK
GW = 104


def _sc_body(idx_hbm, val_hbm, w_hbm, bias_hbm, out_hbm,
             idx_v, val_v, gat_v, out_v, bias_v, drain_v, sem):
    wid = lax.axis_index("s") * NC + lax.axis_index("c")
    pltpu.sync_copy(bias_hbm, bias_v)
    lane = jax.lax.iota(jnp.int32, 16)
    tail_mask = lane >= 12

    def block(blk, carry):
        row0 = wid * ROWS_PER_W + blk * RBLK
        pltpu.sync_copy(idx_hbm.at[pl.ds(row0, RBLK)], idx_v)
        pltpu.sync_copy(val_hbm.at[pl.ds(row0, RBLK)], val_v)

        def issue(r, c):
            pltpu.async_copy(w_hbm.at[0].at[idx_v.at[r, pl.ds(0, F)]],
                             gat_v.at[r, pl.ds(0, F)], sem)
            return c

        lax.fori_loop(0, RBLK, issue, 0)
        pltpu.make_async_copy(w_hbm.at[0].at[pl.ds(0, RBLK * F)],
                              drain_v, sem).wait()

        bvec = bias_v[...]

        def sixteen_rows(rg, c):
            r0 = rg * 16
            outv = jnp.zeros((16,), jnp.float32)
            for k in range(16):
                r = r0 + k
                acc = (gat_v[r, pl.ds(0, 16)] * val_v[r, pl.ds(0, 16)]
                       + gat_v[r, pl.ds(16, 16)] * val_v[r, pl.ds(16, 16)])
                acc2 = (gat_v[r, pl.ds(32, 16)] * val_v[r, pl.ds(32, 16)]
                        + gat_v[r, pl.ds(48, 16)] * val_v[r, pl.ds(48, 16)])
                acc3 = (gat_v[r, pl.ds(64, 16)] * val_v[r, pl.ds(64, 16)]
                        + gat_v[r, pl.ds(80, 16)] * val_v[r, pl.ds(80, 16)])
                tail = jnp.where(tail_mask,
                                 gat_v[r, pl.ds(84, 16)]
                                 * val_v[r, pl.ds(84, 16)], 0.0)
                s = jnp.sum(acc + acc2 + acc3 + tail)
                outv = jnp.where(lane == k, s, outv)
            out_v[pl.ds(r0, 16)] = outv + bvec
            return c

        lax.fori_loop(0, RBLK // 16, sixteen_rows, 0)
        pltpu.sync_copy(out_v, out_hbm.at[pl.ds(row0, RBLK)])
        return carry

    lax.fori_loop(0, NBLK, block, 0)


@jax.jit
def _sc_call(idx_p, val_p, w0, bias16):
    mesh = plsc.VectorSubcoreMesh(core_axis_name="c", subcore_axis_name="s")
    f = pl.kernel(
        _sc_body,
        mesh=mesh,
        out_type=jax.ShapeDtypeStruct((B,), jnp.float32),
        scratch_types=[
            pltpu.VMEM((RBLK, FP), jnp.int32),
            pltpu.VMEM((RBLK, FP), jnp.float32),
            pltpu.VMEM((RBLK, GW), jnp.float32),
            pltpu.VMEM((RBLK,), jnp.float32),
            pltpu.VMEM((16,), jnp.float32),
            pltpu.VMEM((RBLK * F,), jnp.float32),
            pltpu.SemaphoreType.DMA,
        ],
        compiler_params=pltpu.CompilerParams(needs_layout_passes=False),
    )
    return f(idx_p, val_p, w0, bias16)


def kernel(index_list, value_list, W, bias):
    idx_p = jax.lax.dynamic_update_slice(
        jnp.zeros((B, FP), index_list.dtype), index_list, (0, 0))
    val_p = jax.lax.dynamic_update_slice(
        jnp.zeros((B, FP), value_list.dtype), value_list, (0, 0))
    bias16 = jnp.broadcast_to(bias, (16,))
    res = _sc_call(idx_p, val_p, W, bias16)
    return res.reshape(B, 1)


# speedup vs baseline: 1.3460x; 1.1091x over previous
import jax
import jax.numpy as jnp
from jax import lax
from jax.experimental import pallas as pl
from jax.experimental.pallas import tpu as pltpu
from jax.experimental.pallas import tpu_sc as plsc

B = 16384
F = 100
FP = 128
V = 1000000
NC = 2
NS = 16
NW = NC * NS
ROWS_PER_W = B // NW
RBLK = 128
NBLK = ROWS_PER_W // RBLK
GW = 104


def _sc_body(idx_hbm, val_hbm, w_hbm, bias_hbm, out_hbm,
             idx_v, val_v, gat_v, out_v, bias_v, drain_v,
             sem_in0, sem_in1, sem_g0, sem_g1):
    wid = lax.axis_index("s") * NC + lax.axis_index("c")
    row_base = wid * ROWS_PER_W
    pltpu.sync_copy(bias_hbm, bias_v)
    lane = jax.lax.iota(jnp.int32, 16)
    tail_mask = lane >= 12
    sem_in = (sem_in0, sem_in1)
    sem_g = (sem_g0, sem_g1)

    def start_in(blk):
        par = blk & 1
        row0 = row_base + blk * RBLK
        hi = pltpu.async_copy(idx_hbm.at[pl.ds(row0, RBLK)],
                              idx_v.at[par], sem_in[par])
        hv = pltpu.async_copy(val_hbm.at[pl.ds(row0, RBLK)],
                              val_v.at[par], sem_in[par])
        return hi, hv

    def issue_gathers(blk):
        par = blk & 1

        def issue(r, c):
            pltpu.async_copy(
                w_hbm.at[0].at[idx_v.at[par, r, pl.ds(0, F)]],
                gat_v.at[par, r, pl.ds(0, F)], sem_g[par])
            return c

        lax.fori_loop(0, RBLK, issue, 0)

    def drain_gathers(blk):
        pltpu.make_async_copy(w_hbm.at[0].at[pl.ds(0, RBLK * F)],
                              drain_v, sem_g[blk & 1]).wait()

    def compute(blk):
        par = blk & 1
        row0 = row_base + blk * RBLK
        bvec = bias_v[...]

        def sixteen_rows(rg, c):
            r0 = rg * 16
            outv = jnp.zeros((16,), jnp.float32)
            for k in range(16):
                r = r0 + k
                acc = (gat_v[par, r, pl.ds(0, 16)] * val_v[par, r, pl.ds(0, 16)]
                       + gat_v[par, r, pl.ds(16, 16)] * val_v[par, r, pl.ds(16, 16)])
                acc2 = (gat_v[par, r, pl.ds(32, 16)] * val_v[par, r, pl.ds(32, 16)]
                        + gat_v[par, r, pl.ds(48, 16)] * val_v[par, r, pl.ds(48, 16)])
                acc3 = (gat_v[par, r, pl.ds(64, 16)] * val_v[par, r, pl.ds(64, 16)]
                        + gat_v[par, r, pl.ds(80, 16)] * val_v[par, r, pl.ds(80, 16)])
                tail = jnp.where(tail_mask,
                                 gat_v[par, r, pl.ds(84, 16)]
                                 * val_v[par, r, pl.ds(84, 16)], 0.0)
                s = jnp.sum(acc + acc2 + acc3 + tail)
                outv = jnp.where(lane == k, s, outv)
            out_v[pl.ds(r0, 16)] = outv + bvec
            return c

        lax.fori_loop(0, RBLK // 16, sixteen_rows, 0)
        pltpu.sync_copy(out_v, out_hbm.at[pl.ds(row0, RBLK)])

    handles = {0: start_in(0)}
    handles[0][0].wait()
    handles[0][1].wait()
    issue_gathers(0)
    handles[1] = start_in(1)
    for blk in range(NBLK):
        if blk + 1 < NBLK:
            handles[blk + 1][0].wait()
            handles[blk + 1][1].wait()
            issue_gathers(blk + 1)
        drain_gathers(blk)
        compute(blk)
        if blk + 2 < NBLK:
            handles[blk + 2] = start_in(blk + 2)


@jax.jit
def _sc_call(idx_p, val_p, w2d, bias16):
    mesh = plsc.VectorSubcoreMesh(core_axis_name="c", subcore_axis_name="s")
    f = pl.kernel(
        _sc_body,
        mesh=mesh,
        out_type=jax.ShapeDtypeStruct((B,), jnp.float32),
        scratch_types=[
            pltpu.VMEM((2, RBLK, FP), jnp.int32),
            pltpu.VMEM((2, RBLK, FP), jnp.float32),
            pltpu.VMEM((2, RBLK, GW), jnp.float32),
            pltpu.VMEM((RBLK,), jnp.float32),
            pltpu.VMEM((16,), jnp.float32),
            pltpu.VMEM((RBLK * F,), jnp.float32),
            pltpu.SemaphoreType.DMA,
            pltpu.SemaphoreType.DMA,
            pltpu.SemaphoreType.DMA,
            pltpu.SemaphoreType.DMA,
        ],
        compiler_params=pltpu.CompilerParams(needs_layout_passes=False),
    )
    return f(idx_p, val_p, w2d, bias16)


def kernel(index_list, value_list, W, bias):
    idx_p = jax.lax.dynamic_update_slice(
        jnp.zeros((B, FP), index_list.dtype), index_list, (0, 0))
    val_p = jax.lax.dynamic_update_slice(
        jnp.zeros((B, FP), value_list.dtype), value_list, (0, 0))
    bias16 = jnp.broadcast_to(bias, (16,))
    res = _sc_call(idx_p, val_p, W, bias16)
    return res.reshape(B, 1)
